# baseline (speedup 1.0000x reference)
import jax
import jax.numpy as jnp
from jax.experimental import pallas as pl
from jax.experimental.pallas import tpu as pltpu

NMS_THRESH = 0.3
N = 1000
NP = 1024
C = 151
CH = 32


def _nms_argmax_kernel(p_ref, best_ref, pred_ref, m_ref, qt_ref):
    c = pl.program_id(0)

    @pl.when(c == 0)
    def _init():
        best_ref[...] = jnp.full(best_ref.shape, -1.0, jnp.float32)
        pred_ref[...] = jnp.full(pred_ref.shape, 1, jnp.int32)

    p = p_ref[0]
    x1r = p[0:1, :]
    y1r = p[1:2, :]
    x2r = p[2:3, :]
    y2r = p[3:4, :]
    sr = p[4:5, :]
    ar = (x2r - x1r + 1.0) * (y2r - y1r + 1.0)

    q = jnp.concatenate([p[0:4, :], ar, sr, jnp.zeros((2, NP), jnp.float32)],
                        axis=0)
    qt_ref[...] = q.T

    def build(i, cnt0):
        r0 = i * CH
        qc = qt_ref[pl.ds(r0, CH), :]
        x1c = qc[:, 0:1]
        y1c = qc[:, 1:2]
        x2c = qc[:, 2:3]
        y2c = qc[:, 3:4]
        ac = qc[:, 4:5]
        sc = qc[:, 5:6]
        xx1 = jnp.maximum(x1c, x1r)
        yy1 = jnp.maximum(y1c, y1r)
        xx2 = jnp.minimum(x2c, x2r)
        yy2 = jnp.minimum(y2c, y2r)
        w = jnp.maximum(0.0, xx2 - xx1 + 1.0)
        h = jnp.maximum(0.0, yy2 - yy1 + 1.0)
        inter = w * h
        iou = inter / (ac + ar - inter)
        ri = jax.lax.broadcasted_iota(jnp.int32, (CH, NP), 0) + r0
        ci = jax.lax.broadcasted_iota(jnp.int32, (CH, NP), 1)
        dom = (sc > sr) | ((sc == sr) & (ri < ci))
        mb = (iou > NMS_THRESH) & dom
        m_ref[pl.ds(r0, CH), :] = mb.astype(jnp.bfloat16)
        return cnt0 + jnp.sum(mb.astype(jnp.float32), axis=0, keepdims=True)

    cnt0 = jax.lax.fori_loop(0, NP // CH, build,
                             jnp.zeros((1, NP), jnp.float32))

    def step(k):
        cnt = jnp.dot(k.astype(jnp.bfloat16), m_ref[...],
                      preferred_element_type=jnp.float32)
        return (cnt == 0.0).astype(jnp.float32)

    k = (cnt0 == 0.0).astype(jnp.float32)
    k = jax.lax.fori_loop(0, 4, lambda i, kk: step(kk), k)

    def wbody(carry):
        kk, _ = carry
        k2 = step(step(kk))
        return k2, jnp.any(k2 != kk)

    k, _ = jax.lax.while_loop(lambda carry: carry[1], wbody,
                              (k, jnp.bool_(True)))

    val = k * sr
    best = best_ref[0:1, :]
    upd = val > best
    best_ref[0:1, :] = jnp.where(upd, val, best)
    cls = jnp.full((1, NP), c + 1, jnp.int32)
    pred_ref[0:1, :] = jnp.where(upd, cls, pred_ref[0:1, :])


def _relhead_kernel(vr_ref, w_ref, b_ref, out_ref):
    acc = jax.lax.dot_general(
        vr_ref[...], w_ref[...],
        dimension_numbers=(((1,), (1,)), ((), ())),
        preferred_element_type=jnp.float32,
    )
    out_ref[...] = acc + b_ref[...]


@jax.jit
def kernel(obj_logits, vr, boxes_per_cls, W, b):
    probs = jax.nn.softmax(obj_logits, axis=1)

    bT = jnp.transpose(boxes_per_cls, (1, 2, 0))
    bT = jnp.pad(bT, ((0, 0), (0, 0), (0, NP - N)))
    sT = jnp.pad(probs.T[:, None, :], ((0, 0), (0, 0), (0, NP - N)),
                 constant_values=-1.0)
    pad = jnp.zeros((C, 3, NP), jnp.float32)
    packed = jnp.concatenate([bT, sT, pad], axis=1)

    best, preds = pl.pallas_call(
        _nms_argmax_kernel,
        grid=(C - 1,),
        in_specs=[pl.BlockSpec((1, 8, NP), lambda c: (c + 1, 0, 0))],
        out_specs=[pl.BlockSpec((8, NP), lambda c: (0, 0)),
                   pl.BlockSpec((8, NP), lambda c: (0, 0))],
        out_shape=[jax.ShapeDtypeStruct((8, NP), jnp.float32),
                   jax.ShapeDtypeStruct((8, NP), jnp.int32)],
        scratch_shapes=[pltpu.VMEM((NP, NP), jnp.bfloat16),
                        pltpu.VMEM((NP, 8), jnp.float32)],
    )(packed)
    obj_preds = preds[0, :N]

    RB = 400
    rel_dists = pl.pallas_call(
        _relhead_kernel,
        grid=(vr.shape[0] // RB,),
        in_specs=[pl.BlockSpec((RB, vr.shape[1]), lambda i: (i, 0)),
                  pl.BlockSpec(W.shape, lambda i: (0, 0)),
                  pl.BlockSpec((1, W.shape[0]), lambda i: (0, 0))],
        out_specs=pl.BlockSpec((RB, W.shape[0]), lambda i: (i, 0)),
        out_shape=jax.ShapeDtypeStruct((vr.shape[0], W.shape[0]), jnp.float32),
    )(vr, W, b.reshape(1, -1))

    return (obj_logits, obj_preds, rel_dists)


# speedup vs baseline: 1.3605x; 1.3605x over previous
import jax
import jax.numpy as jnp
from jax.experimental import pallas as pl
from jax.experimental.pallas import tpu as pltpu

NMS_THRESH = 0.3
N = 1000
NP = 1024
C = 151
CH = 32
NW = NP // CH


def _nms_argmax_kernel(p_ref, best_ref, pred_ref, m_ref, pk_ref):
    c = pl.program_id(0)

    @pl.when(c == 0)
    def _init():
        best_ref[...] = jnp.full(best_ref.shape, -1.0, jnp.float32)
        pred_ref[...] = jnp.full(pred_ref.shape, 1, jnp.int32)
        ii = jax.lax.broadcasted_iota(jnp.int32, (NP, 2 * NW), 0)
        gg = jax.lax.broadcasted_iota(jnp.int32, (NP, 2 * NW), 1)
        word = jax.lax.shift_right_logical(ii, 5)
        sel = (word == (gg & (NW - 1))) & ((gg >= NW) == ((ii & 16) != 0))
        pw = jax.lax.shift_left(jnp.int32(1), ii & 15)
        pk_ref[...] = jnp.where(sel, pw, 0).astype(jnp.float32)

    p = p_ref[0]
    x1r = p[0:1, :]
    y1r = p[1:2, :]
    x2r = p[2:3, :]
    y2r = p[3:4, :]
    sr = p[4:5, :]
    ar = (x2r - x1r + 1.0) * (y2r - y1r + 1.0)

    q = jnp.concatenate([p[0:4, :], ar, sr, jnp.zeros((2, NP), jnp.float32)],
                        axis=0)
    qT = q.T

    bitcol = jax.lax.shift_left(
        jnp.int32(1),
        jax.lax.broadcasted_iota(jnp.int32, (CH, 1), 0))
    ci = jax.lax.broadcasted_iota(jnp.int32, (CH, NP), 1)
    ri0 = jax.lax.broadcasted_iota(jnp.int32, (CH, NP), 0)

    for g in range(NW):
        r0 = g * CH
        qc = qT[r0:r0 + CH]
        x1c = qc[:, 0:1]
        y1c = qc[:, 1:2]
        x2c = qc[:, 2:3]
        y2c = qc[:, 3:4]
        ac = qc[:, 4:5]
        sc = qc[:, 5:6]
        xx1 = jnp.maximum(x1c, x1r)
        yy1 = jnp.maximum(y1c, y1r)
        xx2 = jnp.minimum(x2c, x2r)
        yy2 = jnp.minimum(y2c, y2r)
        w = jnp.maximum(0.0, xx2 - xx1 + 1.0)
        h = jnp.maximum(0.0, yy2 - yy1 + 1.0)
        inter = w * h
        iou = inter / (ac + ar - inter)
        dom = (sc > sr) | ((sc == sr) & (ri0 + r0 < ci))
        mb = (iou > NMS_THRESH) & dom
        t = jnp.where(mb, bitcol, 0)
        t = t[0:16] | t[16:32]
        t = t[0:8] | t[8:16]
        t = t[0:4] | t[4:8]
        t = t[0:2] | t[2:4]
        m_ref[g:g + 1, :] = t[0:1] | t[1:2]

    def bstep(kb):
        anded = m_ref[...] & kb
        t = anded[0:16] | anded[16:32]
        t = t[0:8] | t[8:16]
        t = t[0:4] | t[4:8]
        t = t[0:2] | t[2:4]
        orr = t[0:1] | t[1:2]
        kf = (orr == 0).astype(jnp.float32)
        packed = jnp.dot(kf, pk_ref[...],
                         preferred_element_type=jnp.float32)
        lo = packed[:, 0:NW].astype(jnp.int32)
        hi = packed[:, NW:2 * NW].astype(jnp.int32)
        kb_new = (lo | jax.lax.shift_left(hi, 16)).T
        return kb_new, kf

    kb = jnp.full((NW, 1), -1, jnp.int32)
    kf = jnp.ones((1, NP), jnp.float32)
    for _ in range(5):
        kb, kf = bstep(kb)

    def wbody(carry):
        kb0, _, _ = carry
        kb1, _ = bstep(kb0)
        kb2, kf2 = bstep(kb1)
        return kb2, kf2, jnp.any(kb2 != kb0)

    kb, kf, _ = jax.lax.while_loop(lambda carry: carry[2], wbody,
                                   (kb, kf, jnp.bool_(True)))

    val = kf * sr
    best = best_ref[0:1, :]
    upd = val > best
    best_ref[0:1, :] = jnp.where(upd, val, best)
    cls = jnp.full((1, NP), c + 1, jnp.int32)
    pred_ref[0:1, :] = jnp.where(upd, cls, pred_ref[0:1, :])


def _relhead_kernel(vr_ref, w_ref, b_ref, out_ref):
    acc = jax.lax.dot_general(
        vr_ref[...], w_ref[...],
        dimension_numbers=(((1,), (1,)), ((), ())),
        preferred_element_type=jnp.float32,
    )
    out_ref[...] = acc + b_ref[...]


@jax.jit
def kernel(obj_logits, vr, boxes_per_cls, W, b):
    probs = jax.nn.softmax(obj_logits, axis=1)

    bT = jnp.transpose(boxes_per_cls, (1, 2, 0))
    bT = jnp.pad(bT, ((0, 0), (0, 0), (0, NP - N)))
    sT = jnp.pad(probs.T[:, None, :], ((0, 0), (0, 0), (0, NP - N)),
                 constant_values=-1.0)
    pad = jnp.zeros((C, 3, NP), jnp.float32)
    packed = jnp.concatenate([bT, sT, pad], axis=1)

    best, preds = pl.pallas_call(
        _nms_argmax_kernel,
        grid=(C - 1,),
        in_specs=[pl.BlockSpec((1, 8, NP), lambda c: (c + 1, 0, 0))],
        out_specs=[pl.BlockSpec((8, NP), lambda c: (0, 0)),
                   pl.BlockSpec((8, NP), lambda c: (0, 0))],
        out_shape=[jax.ShapeDtypeStruct((8, NP), jnp.float32),
                   jax.ShapeDtypeStruct((8, NP), jnp.int32)],
        scratch_shapes=[pltpu.VMEM((NW, NP), jnp.int32),
                        pltpu.VMEM((NP, 2 * NW), jnp.float32)],
    )(packed)
    obj_preds = preds[0, :N]

    RB = 400
    rel_dists = pl.pallas_call(
        _relhead_kernel,
        grid=(vr.shape[0] // RB,),
        in_specs=[pl.BlockSpec((RB, vr.shape[1]), lambda i: (i, 0)),
                  pl.BlockSpec(W.shape, lambda i: (0, 0)),
                  pl.BlockSpec((1, W.shape[0]), lambda i: (0, 0))],
        out_specs=pl.BlockSpec((RB, W.shape[0]), lambda i: (i, 0)),
        out_shape=jax.ShapeDtypeStruct((vr.shape[0], W.shape[0]), jnp.float32),
    )(vr, W, b.reshape(1, -1))

    return (obj_logits, obj_preds, rel_dists)
